# baseline (speedup 1.0000x reference)
import functools
import math

import jax
import jax.numpy as jnp
from jax import lax
from jax.experimental import pallas as pl
from jax.experimental.pallas import tpu as pltpu
from jax.experimental.pallas import tpu_sc as plsc

NC = 2
NS = 16
NW = NC * NS
LANES = 16
CHUNK = 128


@functools.partial(jax.jit, static_argnums=(2, 3))
def _embed(idx, table, n_chunks, scale):
    total = NW * n_chunks * CHUNK
    d = table.shape[1]

    @functools.partial(
        pl.kernel,
        mesh=plsc.VectorSubcoreMesh(core_axis_name="c", subcore_axis_name="s"),
        compiler_params=pltpu.CompilerParams(use_tc_tiling_on_sc=False),
        out_type=jax.ShapeDtypeStruct((total, d), jnp.float32),
        scratch_types=[
            pltpu.VMEM((n_chunks, CHUNK), jnp.int32),
            pltpu.VMEM((CHUNK, d), jnp.float32),
            pltpu.SemaphoreType.DMA,
        ],
    )
    def body(idx_hbm, table_hbm, out_hbm, idx_v, rows_v, sem):
        wid = lax.axis_index("s") * NC + lax.axis_index("c")
        pltpu.sync_copy(idx_hbm.at[pl.ds(wid * n_chunks, n_chunks)], idx_v)
        base = wid * n_chunks * CHUNK

        def chunk_body(j, carry):
            pltpu.async_copy(table_hbm.at[idx_v.at[j]], rows_v, sem).wait()

            def scale_row(r, c2):
                for c in range(d // LANES):
                    s = pl.ds(c * LANES, LANES)
                    rows_v[r, s] = rows_v[r, s] * scale
                return c2

            lax.fori_loop(0, CHUNK, scale_row, 0)
            pltpu.sync_copy(rows_v, out_hbm.at[pl.ds(base + j * CHUNK, CHUNK)])
            return carry

        lax.fori_loop(0, n_chunks, chunk_body, 0)

    return body(idx, table)


def kernel(x, table):
    b, s = x.shape
    v, d = table.shape
    total = b * s
    assert total % (NW * CHUNK) == 0
    n_chunks = total // (NW * CHUNK)
    idx = x.reshape(NW * n_chunks, CHUNK).astype(jnp.int32)
    out = _embed(idx, table, n_chunks, float(math.sqrt(d)))
    return out.reshape(b, s, d)


# speedup vs baseline: 1.1669x; 1.1669x over previous
import functools
import math

import jax
import jax.numpy as jnp
from jax import lax
from jax.experimental import pallas as pl
from jax.experimental.pallas import tpu as pltpu
from jax.experimental.pallas import tpu_sc as plsc

NC = 2
NS = 16
NW = NC * NS
LANES = 16
CHUNK = 128
PADW = 128


@functools.partial(jax.jit, static_argnums=(2, 3))
def _embed(idx, table, n_chunks, scale):
    total = NW * n_chunks * CHUNK
    d = table.shape[1]

    @functools.partial(
        pl.kernel,
        mesh=plsc.VectorSubcoreMesh(core_axis_name="c", subcore_axis_name="s"),
        out_type=jax.ShapeDtypeStruct((total, d), jnp.float32),
        scratch_types=[
            pltpu.VMEM((n_chunks, CHUNK), jnp.int32),
            pltpu.VMEM((CHUNK, d), jnp.float32),
            pltpu.SemaphoreType.DMA,
        ],
    )
    def body(idx_hbm, table_hbm, out_hbm, idx_v, rows_v, sem):
        wid = lax.axis_index("s") * NC + lax.axis_index("c")
        pltpu.sync_copy(idx_hbm.at[pl.ds(wid * n_chunks, n_chunks)], idx_v)
        base = wid * n_chunks * CHUNK

        def chunk_body(j, carry):
            pltpu.async_copy(table_hbm.at[idx_v.at[j]], rows_v, sem).wait()

            def scale_row(r, c2):
                for c in range(64 // LANES):
                    s = pl.ds(c * LANES, LANES)
                    rows_v[r, s] = rows_v[r, s] * scale
                return c2

            lax.fori_loop(0, CHUNK, scale_row, 0)
            pltpu.sync_copy(rows_v, out_hbm.at[pl.ds(base + j * CHUNK, CHUNK)])
            return carry

        lax.fori_loop(0, n_chunks, chunk_body, 0)

    return body(idx, table)


def kernel(x, table):
    b, s = x.shape
    v, d = table.shape
    total = b * s
    assert total % (NW * CHUNK) == 0
    n_chunks = total // (NW * CHUNK)
    idx = x.reshape(NW * n_chunks, CHUNK).astype(jnp.int32)
    tablep = jnp.pad(table, ((0, 0), (0, PADW - d)))
    out = _embed(idx, tablep, n_chunks, float(math.sqrt(d)))
    return out[:, :d].reshape(b, s, d)


# speedup vs baseline: 1.4367x; 1.2312x over previous
import functools
import math

import jax
import jax.numpy as jnp
from jax import lax
from jax.experimental import pallas as pl
from jax.experimental.pallas import tpu as pltpu
from jax.experimental.pallas import tpu_sc as plsc

NC = 2
NS = 16
NW = NC * NS
LANES = 16
CHUNK = 128
PADW = 128


@functools.partial(jax.jit, static_argnums=(2, 3))
def _embed(idx, table, n_chunks, scale):
    total = NW * n_chunks * CHUNK
    d = table.shape[1]

    @functools.partial(
        pl.kernel,
        mesh=plsc.VectorSubcoreMesh(core_axis_name="c", subcore_axis_name="s"),
        out_type=jax.ShapeDtypeStruct((total, d), jnp.float32),
        scratch_types=[
            pltpu.VMEM((n_chunks, CHUNK), jnp.int32),
            pltpu.VMEM((2, CHUNK, d), jnp.float32),
            pltpu.SemaphoreType.DMA,
            pltpu.SemaphoreType.DMA,
            pltpu.SemaphoreType.DMA,
            pltpu.SemaphoreType.DMA,
        ],
    )
    def body(idx_hbm, table_hbm, out_hbm, idx_v, rows_v, si0, si1, so0, so1):
        wid = lax.axis_index("s") * NC + lax.axis_index("c")
        pltpu.sync_copy(idx_hbm.at[pl.ds(wid * n_chunks, n_chunks)], idx_v)
        base = wid * n_chunks * CHUNK
        sin = (si0, si1)
        sout = (so0, so1)

        def gather(j, b):
            pltpu.async_copy(table_hbm.at[idx_v.at[j]], rows_v.at[b], sin[b])

        def wait_gather(j, b):
            pltpu.make_async_copy(
                table_hbm.at[idx_v.at[j]], rows_v.at[b], sin[b]
            ).wait()

        def putout(j, b):
            pltpu.async_copy(
                rows_v.at[b], out_hbm.at[pl.ds(base + j * CHUNK, CHUNK)], sout[b]
            )

        def wait_putout(j, b):
            pltpu.make_async_copy(
                rows_v.at[b], out_hbm.at[pl.ds(base + j * CHUNK, CHUNK)], sout[b]
            ).wait()

        def scale_buf(b):
            def scale_row(r, c2):
                for c in range(64 // LANES):
                    s = pl.ds(c * LANES, LANES)
                    rows_v[b, r, s] = rows_v[b, r, s] * scale
                return c2

            lax.fori_loop(0, CHUNK, scale_row, 0)

        gather(0, 0)

        def pair_body(t, carry):
            j0 = t * 2
            for b in range(2):
                j = j0 + b

                @pl.when(j + 1 < n_chunks)
                def _(j=j, b=b):
                    @pl.when(j >= 1)
                    def _():
                        wait_putout(j - 1, 1 - b)

                    gather(j + 1, 1 - b)

                wait_gather(j, b)
                scale_buf(b)
                putout(j, b)
            return carry

        lax.fori_loop(0, n_chunks // 2, pair_body, 0)
        wait_putout(n_chunks - 2, 0)
        wait_putout(n_chunks - 1, 1)

    return body(idx, table)


def kernel(x, table):
    b, s = x.shape
    v, d = table.shape
    total = b * s
    assert total % (NW * CHUNK) == 0
    n_chunks = total // (NW * CHUNK)
    idx = x.reshape(NW * n_chunks, CHUNK).astype(jnp.int32)
    tablep = jnp.pad(table, ((0, 0), (0, PADW - d)))
    out = _embed(idx, tablep, n_chunks, float(math.sqrt(d)))
    return out[:, :d].reshape(b, s, d)
